# Initial kernel scaffold; baseline (speedup 1.0000x reference)
import jax
import jax.numpy as jnp
from jax.experimental import pallas as pl


def kernel(x, adj_t, params):
    raise NotImplementedError("write your pallas kernel here")


# speedup vs baseline: 21.4146x; 21.4146x over previous
import dataclasses
import functools

import jax
import jax.numpy as jnp
from jax import lax
from jax.experimental import pallas as pl
from jax.experimental.pallas import tpu as pltpu
from jax.experimental.pallas import tpu_sc as plsc

N = 10000
E = 320000
D = 128
H = 128
C = 40
L = 3
NH = 3
EPS = 1e-5

NC = 2
NS = 16
NW = NC * NS
WIN = 128
WPW = 80
EPW = WIN * WPW
EP = NW * EPW
NPAD = 10240
NROW = NPAD // 128
RPT = NPAD // NS
ZR = 32
CH = 2
CW = WPW // CH
BR = 640

_mesh = plsc.VectorSubcoreMesh(core_axis_name="c", subcore_axis_name="s")

_cp = pltpu.CompilerParams()
if "needs_layout_passes" in pltpu.CompilerParams.__dataclass_fields__:
    _cp = dataclasses.replace(_cp, needs_layout_passes=False)


@functools.partial(
    pl.kernel,
    out_type=jax.ShapeDtypeStruct((NC, NROW, 128), jnp.float32),
    mesh=_mesh,
    compiler_params=_cp,
    scratch_types=[
        pltpu.VMEM((WPW, WIN), jnp.int32),
        pltpu.VMEM((NROW, 128), jnp.float32),
        pltpu.VMEM((1, NROW), jnp.int32),
        pltpu.VMEM_SHARED((NROW, 128), jnp.float32),
    ],
)
def _deg_kernel(dst_hbm, row_hbm, out_hbm, dstv, degv, rowv, shared):
    cid = lax.axis_index("c")
    sid = lax.axis_index("s")
    wid = sid * NC + cid
    zero16 = jnp.zeros((16,), jnp.float32)
    one16 = jnp.ones((16,), jnp.float32)

    @pl.loop(0, NROW)
    def _(j):
        for k in range(8):
            degv[j, pl.ds(k * 16, 16)] = zero16

    @pl.when(sid == 0)
    def _():
        pltpu.sync_copy(degv, shared)

    pltpu.sync_copy(row_hbm, rowv)
    pltpu.sync_copy(dst_hbm.at[pl.ds(wid * WPW, WPW)], dstv)

    @pl.loop(0, WPW)
    def _(j):
        for k in range(8):
            idx = dstv[j, pl.ds(k * 16, 16)]
            r = lax.shift_right_logical(idx, 7)
            cc = jnp.bitwise_and(idx, 127)
            plsc.addupdate_scatter(degv, [r, cc], one16)

    plsc.subcore_barrier()
    pltpu.sync_copy(degv, shared.at[rowv.at[0]], add=True)
    plsc.subcore_barrier()

    @pl.when(sid < NROW // 8)
    def _():
        pltpu.sync_copy(shared.at[pl.ds(sid * 8, 8)],
                        out_hbm.at[cid, pl.ds(sid * 8, 8)])


@functools.partial(
    pl.kernel,
    out_type=jax.ShapeDtypeStruct((NC, NPAD, 128), jnp.float32),
    mesh=_mesh,
    scratch_types=[
        pltpu.VMEM((CW, WIN), jnp.int32),
        pltpu.VMEM((CW, WIN), jnp.int32),
        pltpu.VMEM((WIN, 128), jnp.float32),
        pltpu.VMEM((WIN, 128), jnp.float32),
        pltpu.VMEM((ZR, 128), jnp.float32),
        pltpu.VMEM_SHARED((NPAD, 128), jnp.float32),
        pltpu.SemaphoreType.DMA,
    ],
)
def _seg_kernel(hs_hbm, src_hbm, dst_hbm, out_hbm,
                srcv, dstv, buf0, buf1, zv, shared, gsem):
    cid = lax.axis_index("c")
    sid = lax.axis_index("s")
    wid = sid * NC + cid
    base = wid * WPW
    zero16 = jnp.zeros((16,), jnp.float32)

    @pl.loop(0, ZR)
    def _(j):
        for k in range(8):
            zv[j, pl.ds(k * 16, 16)] = zero16

    row0 = sid * RPT

    @pl.loop(0, RPT // ZR)
    def _(i):
        pltpu.sync_copy(zv, shared.at[pl.ds(row0 + i * ZR, ZR)])

    plsc.subcore_barrier()

    def _step(j, cur, nxt):
        pltpu.make_async_copy(hs_hbm.at[srcv.at[j]], cur, gsem).wait()

        @pl.when(j + 1 < CW)
        def _():
            pltpu.async_copy(hs_hbm.at[srcv.at[j + 1]], nxt, gsem)

        pltpu.sync_copy(cur, shared.at[dstv.at[j]], add=True)

    for c in range(CH):
        pltpu.sync_copy(src_hbm.at[pl.ds(base + c * CW, CW)], srcv)
        pltpu.sync_copy(dst_hbm.at[pl.ds(base + c * CW, CW)], dstv)
        pltpu.async_copy(hs_hbm.at[srcv.at[0]], buf0, gsem)

        @pl.loop(0, CW)
        def _(j):
            @pl.when(j % 2 == 0)
            def _():
                _step(j, buf0, buf1)

            @pl.when(j % 2 == 1)
            def _():
                _step(j, buf1, buf0)

    plsc.subcore_barrier()
    pltpu.sync_copy(shared.at[pl.ds(row0, RPT)],
                    out_hbm.at[cid, pl.ds(row0, RPT)])


def _row_mask(i):
    rows = i * BR + lax.broadcasted_iota(jnp.int32, (BR, 1), 0)
    return rows < N


def _scale_mm(xp, w, dinv):
    def body(x_ref, w_ref, dv_ref, o_ref):
        y = jnp.dot(x_ref[...], w_ref[...], preferred_element_type=jnp.float32)
        o_ref[...] = jnp.where(_row_mask(pl.program_id(0)), dv_ref[...] * y, 0.0)

    return pl.pallas_call(
        body,
        grid=(NPAD // BR,),
        in_specs=[
            pl.BlockSpec((BR, 128), lambda i: (i, 0)),
            pl.BlockSpec((128, 128), lambda i: (0, 0)),
            pl.BlockSpec((BR, 1), lambda i: (i, 0)),
        ],
        out_specs=pl.BlockSpec((BR, 128), lambda i: (i, 0)),
        out_shape=jax.ShapeDtypeStruct((NPAD, 128), jnp.float32),
    )(xp, w, dinv)


def _post_mm(p0, p1, hp, dinv, srow, brow, w):
    def body(p0_ref, p1_ref, hp_ref, dv_ref, s_ref, b_ref, w_ref, o_ref):
        agg = dv_ref[...] * (p0_ref[...] + p1_ref[...] + hp_ref[...])
        act = jnp.maximum(agg * s_ref[...] + b_ref[...], 0.0)
        y = jnp.dot(act, w_ref[...], preferred_element_type=jnp.float32)
        o_ref[...] = jnp.where(_row_mask(pl.program_id(0)), dv_ref[...] * y, 0.0)

    return pl.pallas_call(
        body,
        grid=(NPAD // BR,),
        in_specs=[
            pl.BlockSpec((BR, 128), lambda i: (i, 0)),
            pl.BlockSpec((BR, 128), lambda i: (i, 0)),
            pl.BlockSpec((BR, 128), lambda i: (i, 0)),
            pl.BlockSpec((BR, 1), lambda i: (i, 0)),
            pl.BlockSpec((1, 128), lambda i: (0, 0)),
            pl.BlockSpec((1, 128), lambda i: (0, 0)),
            pl.BlockSpec((128, 128), lambda i: (0, 0)),
        ],
        out_specs=pl.BlockSpec((BR, 128), lambda i: (i, 0)),
        out_shape=jax.ShapeDtypeStruct((NPAD, 128), jnp.float32),
    )(p0, p1, hp, dinv, srow, brow, w)


def _post_act(p0, p1, hp, dinv, srow, brow):
    def body(p0_ref, p1_ref, hp_ref, dv_ref, s_ref, b_ref, o_ref):
        agg = dv_ref[...] * (p0_ref[...] + p1_ref[...] + hp_ref[...])
        act = jnp.maximum(agg * s_ref[...] + b_ref[...], 0.0)
        o_ref[...] = jnp.where(_row_mask(pl.program_id(0)), act, 0.0)

    return pl.pallas_call(
        body,
        grid=(NPAD // BR,),
        in_specs=[
            pl.BlockSpec((BR, 128), lambda i: (i, 0)),
            pl.BlockSpec((BR, 128), lambda i: (i, 0)),
            pl.BlockSpec((BR, 128), lambda i: (i, 0)),
            pl.BlockSpec((BR, 1), lambda i: (i, 0)),
            pl.BlockSpec((1, 128), lambda i: (0, 0)),
            pl.BlockSpec((1, 128), lambda i: (0, 0)),
        ],
        out_specs=pl.BlockSpec((BR, 128), lambda i: (i, 0)),
        out_shape=jax.ShapeDtypeStruct((NPAD, 128), jnp.float32),
    )(p0, p1, hp, dinv, srow, brow)


def _heads(act, fg0, fgc, fl, ow, vecs, ob):
    def body(a_ref, fg0_ref, fgc_ref, fl_ref, ow_ref, v_ref, ob_ref,
             og_ref, o0_ref, o1_ref, o2_ref):
        a = a_ref[...]

        def lsm(z):
            m = jnp.max(z, axis=-1, keepdims=True)
            return z - m - jnp.log(jnp.sum(jnp.exp(z - m), axis=-1, keepdims=True))

        def mm(u, w):
            return jnp.dot(u, w, preferred_element_type=jnp.float32)

        xg1 = jnp.maximum(mm(a, fg0_ref[...]) * v_ref[0] + v_ref[3], 0.0)
        w1 = fgc_ref[0]
        xg2 = jnp.maximum(
            (mm(xg1, w1[:128]) + mm(a, w1[128:])) * v_ref[1] + v_ref[4], 0.0)
        w2 = fgc_ref[1]
        xg3 = jnp.maximum(
            (mm(xg2, w2[:128]) + mm(a, w2[128:])) * v_ref[2] + v_ref[5], 0.0)
        og_ref[...] = lsm(mm(xg3, ow_ref[0]) + ob_ref[0])

        outs = (o0_ref, o1_ref, o2_ref)
        xgs = (xg1, xg2, xg3)
        for i in range(NH):
            hl = jnp.maximum(mm(xgs[i], fl_ref[i]) * v_ref[6 + i] + v_ref[9 + i],
                             0.0)
            outs[i][...] = lsm(mm(hl, ow_ref[i + 1]) + ob_ref[i + 1])

    blk = pl.BlockSpec((BR, 128), lambda i: (i, 0))
    full = lambda shape: pl.BlockSpec(shape, lambda i: tuple(0 for _ in shape))
    return pl.pallas_call(
        body,
        grid=(NPAD // BR,),
        in_specs=[
            blk,
            full((128, 128)),
            full((2, 256, 128)),
            full((NH, 128, 128)),
            full((NH + 1, 128, 128)),
            full((12, 128)),
            full((NH + 1, 128)),
        ],
        out_specs=[blk, blk, blk, blk],
        out_shape=[jax.ShapeDtypeStruct((NPAD, 128), jnp.float32)] * 4,
    )(act, fg0, fgc, fl, ow, vecs, ob)


@jax.jit
def kernel(x, adj_t, params):
    src = adj_t[0]
    dst = adj_t[1]
    pad = (jnp.arange(EP - E, dtype=jnp.int32) % (NPAD - N)) + N
    srcp = jnp.concatenate([src, pad]).reshape(EP // WIN, WIN)
    dstp = jnp.concatenate([dst, pad]).reshape(EP // WIN, WIN)
    rowids = jnp.arange(NROW, dtype=jnp.int32).reshape(1, NROW)

    degp = _deg_kernel(dstp, rowids)
    deg = (degp[0] + degp[1]).reshape(NPAD)
    dinv = lax.rsqrt(deg + 1.0).reshape(NPAD, 1)

    isq = 1.0 / jnp.sqrt(1.0 + EPS)
    xp = jnp.pad(x, ((0, NPAD - N), (0, 0)))
    hs = _scale_mm(xp, params["conv_W"][0], dinv)
    act = None
    for l in range(L):
        parts = _seg_kernel(hs, srcp, dstp)
        s = (params["bn_g"][l] * isq).reshape(1, 128)
        b = (params["conv_b"][l] * params["bn_g"][l] * isq
             + params["bn_b"][l]).reshape(1, 128)
        if l < L - 1:
            hs = _post_mm(parts[0], parts[1], hs, dinv, s, b,
                          params["conv_W"][l + 1])
        else:
            act = _post_act(parts[0], parts[1], hs, dinv, s, b)

    sg = [params["bng_g"][i] * isq for i in range(NH)]
    bg = [params["fcg_b"][i] * sg[i] + params["bng_b"][i] for i in range(NH)]
    sl = [params["bnl_g"][i] * isq for i in range(NH)]
    bl = [params["fcl_b"][i] * sl[i] + params["bnl_b"][i] for i in range(NH)]
    vecs = jnp.stack(sg + bg + sl + bl)

    ow = jnp.zeros((NH + 1, 128, 128), jnp.float32)
    ow = ow.at[0, :, :C].set(params["outg_W"])
    ob = jnp.full((NH + 1, 128), -1e30, jnp.float32)
    ob = ob.at[0, :C].set(params["outg_b"])
    for i in range(NH):
        ow = ow.at[i + 1, :, :C].set(params["outl_W"][i])
        ob = ob.at[i + 1, :C].set(params["outl_b"][i])
    fgc = jnp.stack([params["fcg_W"][1], params["fcg_W"][2]])
    fl = jnp.stack(params["fcl_W"])

    og, o0, o1, o2 = _heads(act, params["fcg_W"][0], fgc, fl, ow, vecs, ob)
    return (og[:N, :C], o0[:N, :C], o1[:N, :C], o2[:N, :C])


# speedup vs baseline: 21.6940x; 1.0130x over previous
import dataclasses
import functools

import jax
import jax.numpy as jnp
from jax import lax
from jax.experimental import pallas as pl
from jax.experimental.pallas import tpu as pltpu
from jax.experimental.pallas import tpu_sc as plsc

N = 10000
E = 320000
D = 128
H = 128
C = 40
L = 3
NH = 3
EPS = 1e-5

NC = 2
NS = 16
NW = NC * NS
WIN = 128
WPW = 80
EPW = WIN * WPW
EP = NW * EPW
NPAD = 10240
NROW = NPAD // 128
RPT = NPAD // NS
ZR = 32
CH = 2
CW = WPW // CH
BR = 640

_mesh = plsc.VectorSubcoreMesh(core_axis_name="c", subcore_axis_name="s")

_cp = pltpu.CompilerParams()
if "needs_layout_passes" in pltpu.CompilerParams.__dataclass_fields__:
    _cp = dataclasses.replace(_cp, needs_layout_passes=False)


@functools.partial(
    pl.kernel,
    out_type=jax.ShapeDtypeStruct((NC, NROW, 128), jnp.float32),
    mesh=_mesh,
    compiler_params=_cp,
    scratch_types=[
        pltpu.VMEM((WPW, WIN), jnp.int32),
        pltpu.VMEM((NROW, 128), jnp.float32),
        pltpu.VMEM((1, NROW), jnp.int32),
        pltpu.VMEM_SHARED((NROW, 128), jnp.float32),
    ],
)
def _deg_kernel(dst_hbm, row_hbm, out_hbm, dstv, degv, rowv, shared):
    cid = lax.axis_index("c")
    sid = lax.axis_index("s")
    wid = sid * NC + cid
    zero16 = jnp.zeros((16,), jnp.float32)
    one16 = jnp.ones((16,), jnp.float32)

    @pl.loop(0, NROW)
    def _(j):
        for k in range(8):
            degv[j, pl.ds(k * 16, 16)] = zero16

    @pl.when(sid == 0)
    def _():
        pltpu.sync_copy(degv, shared)

    pltpu.sync_copy(row_hbm, rowv)
    pltpu.sync_copy(dst_hbm.at[pl.ds(wid * WPW, WPW)], dstv)

    @pl.loop(0, WPW)
    def _(j):
        for k in range(8):
            idx = dstv[j, pl.ds(k * 16, 16)]
            r = lax.shift_right_logical(idx, 7)
            cc = jnp.bitwise_and(idx, 127)
            plsc.addupdate_scatter(degv, [r, cc], one16)

    plsc.subcore_barrier()
    pltpu.sync_copy(degv, shared.at[rowv.at[0]], add=True)
    plsc.subcore_barrier()

    @pl.when(sid < NROW // 8)
    def _():
        pltpu.sync_copy(shared.at[pl.ds(sid * 8, 8)],
                        out_hbm.at[cid, pl.ds(sid * 8, 8)])


@functools.partial(
    pl.kernel,
    out_type=jax.ShapeDtypeStruct((NC, NPAD, 128), jnp.float32),
    mesh=_mesh,
    scratch_types=[
        pltpu.VMEM((CW, WIN), jnp.int32),
        pltpu.VMEM((CW, WIN), jnp.int32),
        pltpu.VMEM((WIN, 128), jnp.float32),
        pltpu.VMEM((WIN, 128), jnp.float32),
        pltpu.VMEM((ZR, 128), jnp.float32),
        pltpu.VMEM_SHARED((NPAD, 128), jnp.float32),
        pltpu.SemaphoreType.DMA,
        pltpu.SemaphoreType.DMA,
    ],
)
def _seg_kernel(hs_hbm, src_hbm, dst_hbm, out_hbm,
                srcv, dstv, buf0, buf1, zv, shared, gsem, ssem):
    cid = lax.axis_index("c")
    sid = lax.axis_index("s")
    wid = sid * NC + cid
    base = wid * WPW
    zero16 = jnp.zeros((16,), jnp.float32)

    @pl.loop(0, ZR)
    def _(j):
        for k in range(8):
            zv[j, pl.ds(k * 16, 16)] = zero16

    row0 = sid * RPT

    @pl.loop(0, RPT // ZR)
    def _(i):
        pltpu.sync_copy(zv, shared.at[pl.ds(row0 + i * ZR, ZR)])

    plsc.subcore_barrier()

    def _step(j, cur, nxt):
        pltpu.make_async_copy(hs_hbm.at[srcv.at[j]], cur, gsem).wait()
        pltpu.async_copy(cur, shared.at[dstv.at[j]], ssem, add=True)

        @pl.when(j >= 1)
        def _():
            pltpu.make_async_copy(nxt, shared.at[dstv.at[j]], ssem).wait()

        @pl.when(j + 1 < CW)
        def _():
            pltpu.async_copy(hs_hbm.at[srcv.at[j + 1]], nxt, gsem)

    for c in range(CH):
        pltpu.sync_copy(src_hbm.at[pl.ds(base + c * CW, CW)], srcv)
        pltpu.sync_copy(dst_hbm.at[pl.ds(base + c * CW, CW)], dstv)
        pltpu.async_copy(hs_hbm.at[srcv.at[0]], buf0, gsem)

        @pl.loop(0, CW)
        def _(j):
            @pl.when(j % 2 == 0)
            def _():
                _step(j, buf0, buf1)

            @pl.when(j % 2 == 1)
            def _():
                _step(j, buf1, buf0)

        pltpu.make_async_copy(buf0, shared.at[dstv.at[CW - 1]], ssem).wait()

    plsc.subcore_barrier()
    pltpu.sync_copy(shared.at[pl.ds(row0, RPT)],
                    out_hbm.at[cid, pl.ds(row0, RPT)])


def _row_mask(i):
    rows = i * BR + lax.broadcasted_iota(jnp.int32, (BR, 1), 0)
    return rows < N


def _scale_mm(xp, w, dinv):
    def body(x_ref, w_ref, dv_ref, o_ref):
        y = jnp.dot(x_ref[...], w_ref[...], preferred_element_type=jnp.float32)
        o_ref[...] = jnp.where(_row_mask(pl.program_id(0)), dv_ref[...] * y, 0.0)

    return pl.pallas_call(
        body,
        grid=(NPAD // BR,),
        in_specs=[
            pl.BlockSpec((BR, 128), lambda i: (i, 0)),
            pl.BlockSpec((128, 128), lambda i: (0, 0)),
            pl.BlockSpec((BR, 1), lambda i: (i, 0)),
        ],
        out_specs=pl.BlockSpec((BR, 128), lambda i: (i, 0)),
        out_shape=jax.ShapeDtypeStruct((NPAD, 128), jnp.float32),
    )(xp, w, dinv)


def _post_mm(p0, p1, hp, dinv, srow, brow, w):
    def body(p0_ref, p1_ref, hp_ref, dv_ref, s_ref, b_ref, w_ref, o_ref):
        agg = dv_ref[...] * (p0_ref[...] + p1_ref[...] + hp_ref[...])
        act = jnp.maximum(agg * s_ref[...] + b_ref[...], 0.0)
        y = jnp.dot(act, w_ref[...], preferred_element_type=jnp.float32)
        o_ref[...] = jnp.where(_row_mask(pl.program_id(0)), dv_ref[...] * y, 0.0)

    return pl.pallas_call(
        body,
        grid=(NPAD // BR,),
        in_specs=[
            pl.BlockSpec((BR, 128), lambda i: (i, 0)),
            pl.BlockSpec((BR, 128), lambda i: (i, 0)),
            pl.BlockSpec((BR, 128), lambda i: (i, 0)),
            pl.BlockSpec((BR, 1), lambda i: (i, 0)),
            pl.BlockSpec((1, 128), lambda i: (0, 0)),
            pl.BlockSpec((1, 128), lambda i: (0, 0)),
            pl.BlockSpec((128, 128), lambda i: (0, 0)),
        ],
        out_specs=pl.BlockSpec((BR, 128), lambda i: (i, 0)),
        out_shape=jax.ShapeDtypeStruct((NPAD, 128), jnp.float32),
    )(p0, p1, hp, dinv, srow, brow, w)


HBR = 400


def _heads(p0, p1, hp, dinv, srow, brow, fg0, fgc, fl, ow, vecs, ob):
    def body(p0_ref, p1_ref, hp_ref, dv_ref, s_ref, b_ref,
             fg0_ref, fgc_ref, fl_ref, ow_ref, v_ref, ob_ref,
             og_ref, o0_ref, o1_ref, o2_ref):
        agg = dv_ref[...] * (p0_ref[...] + p1_ref[...] + hp_ref[...])
        a = jnp.maximum(agg * s_ref[...] + b_ref[...], 0.0)

        def lsm(z):
            m = jnp.max(z, axis=-1, keepdims=True)
            return z - m - jnp.log(jnp.sum(jnp.exp(z - m), axis=-1, keepdims=True))

        def mm(u, w):
            return jnp.dot(u, w, preferred_element_type=jnp.float32)

        xg1 = jnp.maximum(mm(a, fg0_ref[...]) * v_ref[0] + v_ref[3], 0.0)
        w1 = fgc_ref[0]
        xg2 = jnp.maximum(
            (mm(xg1, w1[:128]) + mm(a, w1[128:])) * v_ref[1] + v_ref[4], 0.0)
        w2 = fgc_ref[1]
        xg3 = jnp.maximum(
            (mm(xg2, w2[:128]) + mm(a, w2[128:])) * v_ref[2] + v_ref[5], 0.0)
        og_ref[...] = lsm(mm(xg3, ow_ref[0]) + ob_ref[0])[:, :C]

        outs = (o0_ref, o1_ref, o2_ref)
        xgs = (xg1, xg2, xg3)
        for i in range(NH):
            hl = jnp.maximum(mm(xgs[i], fl_ref[i]) * v_ref[6 + i] + v_ref[9 + i],
                             0.0)
            outs[i][...] = lsm(mm(hl, ow_ref[i + 1]) + ob_ref[i + 1])[:, :C]

    blk = pl.BlockSpec((HBR, 128), lambda i: (i, 0))
    oblk = pl.BlockSpec((HBR, C), lambda i: (i, 0))
    full = lambda shape: pl.BlockSpec(shape, lambda i: tuple(0 for _ in shape))
    return pl.pallas_call(
        body,
        grid=(N // HBR,),
        in_specs=[
            blk,
            blk,
            blk,
            pl.BlockSpec((HBR, 1), lambda i: (i, 0)),
            full((1, 128)),
            full((1, 128)),
            full((128, 128)),
            full((2, 256, 128)),
            full((NH, 128, 128)),
            full((NH + 1, 128, 128)),
            full((12, 128)),
            full((NH + 1, 128)),
        ],
        out_specs=[oblk, oblk, oblk, oblk],
        out_shape=[jax.ShapeDtypeStruct((N, C), jnp.float32)] * 4,
    )(p0, p1, hp, dinv, srow, brow, fg0, fgc, fl, ow, vecs, ob)


@jax.jit
def kernel(x, adj_t, params):
    src = adj_t[0]
    dst = adj_t[1]
    pad = (jnp.arange(EP - E, dtype=jnp.int32) % (NPAD - N)) + N
    srcp = jnp.concatenate([src, pad]).reshape(EP // WIN, WIN)
    dstp = jnp.concatenate([dst, pad]).reshape(EP // WIN, WIN)
    rowids = jnp.arange(NROW, dtype=jnp.int32).reshape(1, NROW)

    degp = _deg_kernel(dstp, rowids)
    deg = (degp[0] + degp[1]).reshape(NPAD)
    dinv = lax.rsqrt(deg + 1.0).reshape(NPAD, 1)

    isq = 1.0 / jnp.sqrt(1.0 + EPS)
    xp = jnp.pad(x, ((0, NPAD - N), (0, 0)))
    hs = _scale_mm(xp, params["conv_W"][0], dinv)
    slast = blast = parts = None
    for l in range(L):
        parts = _seg_kernel(hs, srcp, dstp)
        s = (params["bn_g"][l] * isq).reshape(1, 128)
        b = (params["conv_b"][l] * params["bn_g"][l] * isq
             + params["bn_b"][l]).reshape(1, 128)
        if l < L - 1:
            hs = _post_mm(parts[0], parts[1], hs, dinv, s, b,
                          params["conv_W"][l + 1])
        else:
            slast, blast = s, b

    sg = [params["bng_g"][i] * isq for i in range(NH)]
    bg = [params["fcg_b"][i] * sg[i] + params["bng_b"][i] for i in range(NH)]
    sl = [params["bnl_g"][i] * isq for i in range(NH)]
    bl = [params["fcl_b"][i] * sl[i] + params["bnl_b"][i] for i in range(NH)]
    vecs = jnp.stack(sg + bg + sl + bl)

    ow = jnp.zeros((NH + 1, 128, 128), jnp.float32)
    ow = ow.at[0, :, :C].set(params["outg_W"])
    ob = jnp.full((NH + 1, 128), -1e30, jnp.float32)
    ob = ob.at[0, :C].set(params["outg_b"])
    for i in range(NH):
        ow = ow.at[i + 1, :, :C].set(params["outl_W"][i])
        ob = ob.at[i + 1, :C].set(params["outl_b"][i])
    fgc = jnp.stack([params["fcg_W"][1], params["fcg_W"][2]])
    fl = jnp.stack(params["fcl_W"])

    og, o0, o1, o2 = _heads(parts[0], parts[1], hs, dinv, slast, blast,
                            params["fcg_W"][0], fgc, fl, ow, vecs, ob)
    return (og, o0, o1, o2)


# speedup vs baseline: 23.3868x; 1.0780x over previous
import dataclasses
import functools

import jax
import jax.numpy as jnp
from jax import lax
from jax.experimental import pallas as pl
from jax.experimental.pallas import tpu as pltpu
from jax.experimental.pallas import tpu_sc as plsc

N = 10000
E = 320000
D = 128
H = 128
C = 40
L = 3
NH = 3
EPS = 1e-5

NC = 2
NS = 16
NW = NC * NS
WIN = 128
WPW = 80
EPW = WIN * WPW
EP = NW * EPW
NPAD = 10240
NROW = NPAD // 128
RPT = NPAD // NS
ZR = 32
CH = 2
CW = WPW // CH
BR = 1024

_mesh = plsc.VectorSubcoreMesh(core_axis_name="c", subcore_axis_name="s")

_cp = pltpu.CompilerParams()
if "needs_layout_passes" in pltpu.CompilerParams.__dataclass_fields__:
    _cp = dataclasses.replace(_cp, needs_layout_passes=False)


@functools.partial(
    pl.kernel,
    out_type=jax.ShapeDtypeStruct((NC, NROW, 128), jnp.float32),
    mesh=_mesh,
    compiler_params=_cp,
    scratch_types=[
        pltpu.VMEM((WPW, WIN), jnp.int32),
        pltpu.VMEM((NROW, 128), jnp.float32),
        pltpu.VMEM((1, NROW), jnp.int32),
        pltpu.VMEM_SHARED((NROW, 128), jnp.float32),
    ],
)
def _deg_kernel(dst_hbm, row_hbm, out_hbm, dstv, degv, rowv, shared):
    cid = lax.axis_index("c")
    sid = lax.axis_index("s")
    wid = sid * NC + cid
    zero16 = jnp.zeros((16,), jnp.float32)
    one16 = jnp.ones((16,), jnp.float32)

    @pl.loop(0, NROW)
    def _(j):
        for k in range(8):
            degv[j, pl.ds(k * 16, 16)] = zero16

    @pl.when(sid == 0)
    def _():
        pltpu.sync_copy(degv, shared)

    pltpu.sync_copy(row_hbm, rowv)
    pltpu.sync_copy(dst_hbm.at[pl.ds(wid * WPW, WPW)], dstv)

    @pl.loop(0, WPW)
    def _(j):
        for k in range(8):
            idx = dstv[j, pl.ds(k * 16, 16)]
            r = lax.shift_right_logical(idx, 7)
            cc = jnp.bitwise_and(idx, 127)
            plsc.addupdate_scatter(degv, [r, cc], one16)

    plsc.subcore_barrier()
    pltpu.sync_copy(degv, shared.at[rowv.at[0]], add=True)
    plsc.subcore_barrier()

    @pl.when(sid < NROW // 8)
    def _():
        pltpu.sync_copy(shared.at[pl.ds(sid * 8, 8)],
                        out_hbm.at[cid, pl.ds(sid * 8, 8)])


@functools.partial(
    pl.kernel,
    out_type=[jax.ShapeDtypeStruct((NPAD, 128), jnp.float32),
              jax.ShapeDtypeStruct((NPAD, 128), jnp.float32)],
    mesh=_mesh,
    scratch_types=[
        pltpu.VMEM((CW, WIN), jnp.int32),
        pltpu.VMEM((CW, WIN), jnp.int32),
        pltpu.VMEM((WIN, 128), jnp.float32),
        pltpu.VMEM((WIN, 128), jnp.float32),
        pltpu.VMEM((ZR, 128), jnp.float32),
        pltpu.VMEM_SHARED((NPAD, 128), jnp.float32),
        pltpu.SemaphoreType.DMA,
        pltpu.SemaphoreType.DMA,
    ],
)
def _seg_kernel(hs_hbm, src_hbm, dst_hbm, out0_hbm, out1_hbm,
                srcv, dstv, buf0, buf1, zv, shared, gsem, ssem):
    cid = lax.axis_index("c")
    sid = lax.axis_index("s")
    wid = sid * NC + cid
    base = wid * WPW
    zero16 = jnp.zeros((16,), jnp.float32)

    @pl.loop(0, ZR)
    def _(j):
        for k in range(8):
            zv[j, pl.ds(k * 16, 16)] = zero16

    row0 = sid * RPT

    @pl.loop(0, RPT // ZR)
    def _(i):
        pltpu.sync_copy(zv, shared.at[pl.ds(row0 + i * ZR, ZR)])

    plsc.subcore_barrier()

    def _step(j, cur, nxt):
        pltpu.make_async_copy(hs_hbm.at[srcv.at[j]], cur, gsem).wait()
        pltpu.async_copy(cur, shared.at[dstv.at[j]], ssem, add=True)

        @pl.when(j >= 1)
        def _():
            pltpu.make_async_copy(nxt, shared.at[dstv.at[j]], ssem).wait()

        @pl.when(j + 1 < CW)
        def _():
            pltpu.async_copy(hs_hbm.at[srcv.at[j + 1]], nxt, gsem)

    for c in range(CH):
        pltpu.sync_copy(src_hbm.at[pl.ds(base + c * CW, CW)], srcv)
        pltpu.sync_copy(dst_hbm.at[pl.ds(base + c * CW, CW)], dstv)
        pltpu.async_copy(hs_hbm.at[srcv.at[0]], buf0, gsem)

        @pl.loop(0, CW)
        def _(j):
            @pl.when(j % 2 == 0)
            def _():
                _step(j, buf0, buf1)

            @pl.when(j % 2 == 1)
            def _():
                _step(j, buf1, buf0)

        pltpu.make_async_copy(buf0, shared.at[dstv.at[CW - 1]], ssem).wait()

    plsc.subcore_barrier()

    @pl.when(cid == 0)
    def _():
        pltpu.sync_copy(shared.at[pl.ds(row0, RPT)],
                        out0_hbm.at[pl.ds(row0, RPT)])

    @pl.when(cid == 1)
    def _():
        pltpu.sync_copy(shared.at[pl.ds(row0, RPT)],
                        out1_hbm.at[pl.ds(row0, RPT)])


def _row_mask(i):
    rows = i * BR + lax.broadcasted_iota(jnp.int32, (BR, 1), 0)
    return rows < N


def _scale_mm(xp, w, dinv):
    def body(x_ref, w_ref, dv_ref, o_ref):
        y = jnp.dot(x_ref[...].astype(jnp.bfloat16), w_ref[...],
                    preferred_element_type=jnp.float32)
        o_ref[...] = jnp.where(_row_mask(pl.program_id(0)), dv_ref[...] * y, 0.0)

    return pl.pallas_call(
        body,
        grid=(NPAD // BR,),
        in_specs=[
            pl.BlockSpec((BR, 128), lambda i: (i, 0)),
            pl.BlockSpec((128, 128), lambda i: (0, 0)),
            pl.BlockSpec((BR, 1), lambda i: (i, 0)),
        ],
        out_specs=pl.BlockSpec((BR, 128), lambda i: (i, 0)),
        out_shape=jax.ShapeDtypeStruct((NPAD, 128), jnp.float32),
    )(xp, w, dinv)


def _post_mm(p0, p1, hp, dinv, srow, brow, w):
    def body(p0_ref, p1_ref, hp_ref, dv_ref, s_ref, b_ref, w_ref, o_ref):
        agg = dv_ref[...] * (p0_ref[...] + p1_ref[...] + hp_ref[...])
        act = jnp.maximum(agg * s_ref[...] + b_ref[...], 0.0)
        y = jnp.dot(act.astype(jnp.bfloat16), w_ref[...],
                    preferred_element_type=jnp.float32)
        o_ref[...] = jnp.where(_row_mask(pl.program_id(0)), dv_ref[...] * y, 0.0)

    return pl.pallas_call(
        body,
        grid=(NPAD // BR,),
        in_specs=[
            pl.BlockSpec((BR, 128), lambda i: (i, 0)),
            pl.BlockSpec((BR, 128), lambda i: (i, 0)),
            pl.BlockSpec((BR, 128), lambda i: (i, 0)),
            pl.BlockSpec((BR, 1), lambda i: (i, 0)),
            pl.BlockSpec((1, 128), lambda i: (0, 0)),
            pl.BlockSpec((1, 128), lambda i: (0, 0)),
            pl.BlockSpec((128, 128), lambda i: (0, 0)),
        ],
        out_specs=pl.BlockSpec((BR, 128), lambda i: (i, 0)),
        out_shape=jax.ShapeDtypeStruct((NPAD, 128), jnp.float32),
    )(p0, p1, hp, dinv, srow, brow, w)


HBR = 1000


def _heads(p0, p1, hp, dinv, srow, brow, fg0, fgc, fl, ow, vecs, ob):
    def body(p0_ref, p1_ref, hp_ref, dv_ref, s_ref, b_ref,
             fg0_ref, fgc_ref, fl_ref, ow_ref, v_ref, ob_ref,
             og_ref, o0_ref, o1_ref, o2_ref):
        agg = dv_ref[...] * (p0_ref[...] + p1_ref[...] + hp_ref[...])
        a = jnp.maximum(agg * s_ref[...] + b_ref[...], 0.0)

        def lsm(z):
            m = jnp.max(z, axis=-1, keepdims=True)
            return z - m - jnp.log(jnp.sum(jnp.exp(z - m), axis=-1, keepdims=True))

        def mm(u, w):
            return jnp.dot(u.astype(jnp.bfloat16), w,
                           preferred_element_type=jnp.float32)

        xg1 = jnp.maximum(mm(a, fg0_ref[...]) * v_ref[0] + v_ref[3], 0.0)
        w1 = fgc_ref[0]
        xg2 = jnp.maximum(
            (mm(xg1, w1[:128]) + mm(a, w1[128:])) * v_ref[1] + v_ref[4], 0.0)
        w2 = fgc_ref[1]
        xg3 = jnp.maximum(
            (mm(xg2, w2[:128]) + mm(a, w2[128:])) * v_ref[2] + v_ref[5], 0.0)
        og_ref[...] = lsm(mm(xg3, ow_ref[0]) + ob_ref[0])[:, :C]

        outs = (o0_ref, o1_ref, o2_ref)
        xgs = (xg1, xg2, xg3)
        for i in range(NH):
            hl = jnp.maximum(mm(xgs[i], fl_ref[i]) * v_ref[6 + i] + v_ref[9 + i],
                             0.0)
            outs[i][...] = lsm(mm(hl, ow_ref[i + 1]) + ob_ref[i + 1])[:, :C]

    blk = pl.BlockSpec((HBR, 128), lambda i: (i, 0))
    oblk = pl.BlockSpec((HBR, C), lambda i: (i, 0))
    full = lambda shape: pl.BlockSpec(shape, lambda i: tuple(0 for _ in shape))
    return pl.pallas_call(
        body,
        grid=(N // HBR,),
        in_specs=[
            blk,
            blk,
            blk,
            pl.BlockSpec((HBR, 1), lambda i: (i, 0)),
            full((1, 128)),
            full((1, 128)),
            full((128, 128)),
            full((2, 256, 128)),
            full((NH, 128, 128)),
            full((NH + 1, 128, 128)),
            full((12, 128)),
            full((NH + 1, 128)),
        ],
        out_specs=[oblk, oblk, oblk, oblk],
        out_shape=[jax.ShapeDtypeStruct((N, C), jnp.float32)] * 4,
    )(p0, p1, hp, dinv, srow, brow, fg0, fgc, fl, ow, vecs, ob)


@jax.jit
def kernel(x, adj_t, params):
    src = adj_t[0]
    dst = adj_t[1]
    pad = (jnp.arange(EP - E, dtype=jnp.int32) % (NPAD - N)) + N
    srcp = jnp.concatenate([src, pad]).reshape(EP // WIN, WIN)
    dstp = jnp.concatenate([dst, pad]).reshape(EP // WIN, WIN)
    rowids = jnp.arange(NROW, dtype=jnp.int32).reshape(1, NROW)

    degp = _deg_kernel(dstp, rowids)
    deg = (degp[0] + degp[1]).reshape(NPAD)
    dinv = lax.rsqrt(deg + 1.0).reshape(NPAD, 1)

    isq = 1.0 / jnp.sqrt(1.0 + EPS)
    bf = lambda a: a.astype(jnp.bfloat16)
    xp = jnp.pad(x, ((0, NPAD - N), (0, 0)))
    hs = _scale_mm(xp, bf(params["conv_W"][0]), dinv)
    slast = blast = parts = None
    for l in range(L):
        parts = _seg_kernel(hs, srcp, dstp)
        s = (params["bn_g"][l] * isq).reshape(1, 128)
        b = (params["conv_b"][l] * params["bn_g"][l] * isq
             + params["bn_b"][l]).reshape(1, 128)
        if l < L - 1:
            hs = _post_mm(parts[0], parts[1], hs, dinv, s, b,
                          bf(params["conv_W"][l + 1]))
        else:
            slast, blast = s, b

    sg = [params["bng_g"][i] * isq for i in range(NH)]
    bg = [params["fcg_b"][i] * sg[i] + params["bng_b"][i] for i in range(NH)]
    sl = [params["bnl_g"][i] * isq for i in range(NH)]
    bl = [params["fcl_b"][i] * sl[i] + params["bnl_b"][i] for i in range(NH)]
    vecs = jnp.stack(sg + bg + sl + bl)

    ow = jnp.zeros((NH + 1, 128, 128), jnp.float32)
    ow = ow.at[0, :, :C].set(params["outg_W"])
    ob = jnp.full((NH + 1, 128), -1e30, jnp.float32)
    ob = ob.at[0, :C].set(params["outg_b"])
    for i in range(NH):
        ow = ow.at[i + 1, :, :C].set(params["outl_W"][i])
        ob = ob.at[i + 1, :C].set(params["outl_b"][i])
    fgc = jnp.stack([params["fcg_W"][1], params["fcg_W"][2]])
    fl = jnp.stack(params["fcl_W"])

    og, o0, o1, o2 = _heads(parts[0], parts[1], hs, dinv, slast, blast,
                            bf(params["fcg_W"][0]), bf(fgc), bf(fl), bf(ow),
                            vecs, ob)
    return (og, o0, o1, o2)
